# baseline (speedup 1.0000x reference)
import jax
import jax.numpy as jnp
from jax.experimental import pallas as pl

_J = 100
_M = 20
_N = _J + _M
_H = 3
_F0 = 16
_ED = 128
_SP = 128
_B = 2


def _lrelu(x, s):
    return jnp.maximum(x, s * x)


def _gat(ftall, al, ar, ae, we, b, F, penalty, guard_row, Tm, ones_col):
    acc = jnp.zeros((_SP, F), jnp.float32)
    for h in range(_H):
        fth = ftall[:, h * F:(h + 1) * F]
        fthb = fth.astype(jnp.bfloat16)
        alh = al[h:h + 1, :]
        arh = ar[h:h + 1, :]
        weh = we[h:h + 1, :]
        el = jax.lax.dot_general(fth, alh, (((1,), (1,)), ((), ())),
                                 preferred_element_type=jnp.float32)
        er = jax.lax.dot_general(arh, fth, (((1,), (1,)), ((), ())),
                                 preferred_element_type=jnp.float32)
        eec = jnp.sum(weh * ae[h:h + 1, :])
        lg = _lrelu(el + er + Tm * eec, 0.2) + penalty
        mx = jnp.max(lg, axis=0, keepdims=True)
        ex = jnp.exp(lg - mx)
        den = jnp.sum(ex, axis=0, keepdims=True)
        alpha = ex * (guard_row / den)
        ab = alpha.astype(jnp.bfloat16)
        atb = (alpha * Tm).astype(jnp.bfloat16)
        outh = jax.lax.dot_general(ab, fthb, (((0,), (0,)), ((), ())),
                                   preferred_element_type=jnp.float32)
        eagg = jax.lax.dot_general(atb, ones_col, (((0,), (0,)), ((), ())),
                                   preferred_element_type=jnp.float32)
        outh = outh + eagg * weh + b[h:h + 1, :]
        acc = acc + _lrelu(outh, 0.01)
    return acc * (1.0 / _H)


def _gnn_body(feat_ref, g_ref, t_ref, lng_ref, lnb_ref, w0_ref, al0_ref,
              ar0_ref, ae0_ref, we0_ref, b0_ref, w1_ref, al1_ref, ar1_ref,
              ae1_ref, we1_ref, b1_ref, out_ref):
    lane = jax.lax.broadcasted_iota(jnp.int32, (_SP, 8), 1)
    feat_on = lane < 5
    ones_col = jnp.ones((_SP, 1), jnp.bfloat16)
    for gix in range(_B):
        f = feat_ref[gix]
        fm = jnp.where(feat_on, f, 0.0)
        mu = jnp.sum(fm, axis=1, keepdims=True) * 0.2
        var = jnp.sum(jnp.where(feat_on, (fm - mu) ** 2, 0.0),
                      axis=1, keepdims=True) * 0.2
        nf = (fm - mu) * jax.lax.rsqrt(var + 1e-5) * lng_ref[...] \
            + lnb_ref[...]
        nf = jnp.where(feat_on, nf, 0.0)

        g = g_ref[gix]
        penalty = (g - 1.0) * 1e9
        Tm = t_ref[gix]
        deg = jnp.sum(g, axis=0, keepdims=True)
        guard_row = jnp.where(deg > 0, 1.0, 0.0)

        ft0 = jnp.dot(nf, w0_ref[...], preferred_element_type=jnp.float32)
        h0 = _gat(ft0, al0_ref[...], ar0_ref[...], ae0_ref[...],
                  we0_ref[...], b0_ref[...], _F0, penalty, guard_row,
                  Tm, ones_col)
        ft1 = jnp.dot(h0, w1_ref[...], preferred_element_type=jnp.float32)
        h1 = _gat(ft1, al1_ref[...], ar1_ref[...], ae1_ref[...],
                  we1_ref[...], b1_ref[...], _ED, penalty, guard_row,
                  Tm, ones_col)
        out_ref[gix] = h1.astype(jnp.bfloat16)


def _final_body(x_ref, w_ref, b_ref, out_ref):
    k = pl.program_id(0)
    acc = jnp.zeros((x_ref.shape[0], _ED), jnp.float32)
    for n in range(8):
        acc = acc + jnp.dot(x_ref[:, n, :], w_ref[n],
                            preferred_element_type=jnp.float32)

    @pl.when(k == 0)
    def _():
        out_ref[...] = acc

    @pl.when(k > 0)
    def _():
        out_ref[...] = out_ref[...] + acc

    @pl.when(k == (_N // 8) - 1)
    def _():
        out_ref[...] = _lrelu(out_ref[...] + b_ref[...], 0.01)


def kernel(Graph, norm_h, norm_L, norm_W, norm_P, norm_N, T, ln_g, ln_b,
           W0, We0, al0, ar0, ae0, b0, W1, We1, al1, ar1, ae1, b1, Wl, bl):
    bs = Graph.shape[0]
    G = Graph.reshape(bs, _J, _N)
    Gp = jnp.pad(G, ((0, 0), (0, _SP - _J), (0, _SP - _N)))
    Tp = jnp.pad(T, ((0, 0), (0, _SP - _J), (0, _SP - _J)))
    other = jnp.concatenate([norm_W, norm_P, norm_N], axis=1)
    jobf = jnp.concatenate(
        [norm_h[:, :, None], norm_L[:, :, None],
         jnp.broadcast_to(other[:, None, :], (bs, _J, 3))], axis=2)
    feats = jnp.pad(jobf, ((0, 0), (0, _SP - _J), (0, 3)))
    lng = jnp.pad(ln_g, (0, 3)).reshape(1, 8)
    lnb = jnp.pad(ln_b, (0, 3)).reshape(1, 8)
    w0 = jnp.pad(W0, ((0, 3), (0, 0)))
    we0 = We0.reshape(_H, _F0)
    b0r = b0.reshape(_H, _F0)
    we1 = We1.reshape(_H, _ED)
    b1r = b1.reshape(_H, _ED)

    h1 = pl.pallas_call(
        _gnn_body,
        grid=(bs // _B,),
        in_specs=[
            pl.BlockSpec((_B, _SP, 8), lambda i: (i, 0, 0)),
            pl.BlockSpec((_B, _SP, _SP), lambda i: (i, 0, 0)),
            pl.BlockSpec((_B, _SP, _SP), lambda i: (i, 0, 0)),
            pl.BlockSpec((1, 8), lambda i: (0, 0)),
            pl.BlockSpec((1, 8), lambda i: (0, 0)),
            pl.BlockSpec((8, _H * _F0), lambda i: (0, 0)),
            pl.BlockSpec((_H, _F0), lambda i: (0, 0)),
            pl.BlockSpec((_H, _F0), lambda i: (0, 0)),
            pl.BlockSpec((_H, _F0), lambda i: (0, 0)),
            pl.BlockSpec((_H, _F0), lambda i: (0, 0)),
            pl.BlockSpec((_H, _F0), lambda i: (0, 0)),
            pl.BlockSpec((_F0, _H * _ED), lambda i: (0, 0)),
            pl.BlockSpec((_H, _ED), lambda i: (0, 0)),
            pl.BlockSpec((_H, _ED), lambda i: (0, 0)),
            pl.BlockSpec((_H, _ED), lambda i: (0, 0)),
            pl.BlockSpec((_H, _ED), lambda i: (0, 0)),
            pl.BlockSpec((_H, _ED), lambda i: (0, 0)),
        ],
        out_specs=pl.BlockSpec((_B, _SP, _SP), lambda i: (i, 0, 0)),
        out_shape=jax.ShapeDtypeStruct((bs, _SP, _SP), jnp.bfloat16),
    )(feats, Gp, Tp, lng, lnb, w0, al0, ar0, ae0, we0, b0r,
      W1, al1, ar1, ae1, we1, b1r)

    Wlr = Wl.reshape(_N, _ED, _ED).astype(jnp.bfloat16)
    out = pl.pallas_call(
        _final_body,
        grid=(_N // 8,),
        in_specs=[
            pl.BlockSpec((bs, 8, _ED), lambda k: (0, k, 0)),
            pl.BlockSpec((8, _ED, _ED), lambda k: (k, 0, 0)),
            pl.BlockSpec((1, _ED), lambda k: (0, 0)),
        ],
        out_specs=pl.BlockSpec((bs, _ED), lambda k: (0, 0)),
        out_shape=jax.ShapeDtypeStruct((bs, _ED), jnp.float32),
    )(h1, Wlr, bl.reshape(1, _ED))
    return out


# speedup vs baseline: 1.0194x; 1.0194x over previous
import jax
import jax.numpy as jnp
from jax.experimental import pallas as pl

_J = 100
_M = 20
_N = _J + _M
_H = 3
_F0 = 16
_ED = 128
_SP = 128
_B = 2


def _lrelu(x, s):
    return jnp.maximum(x, s * x)


def _gat(ftall, al, ar, ae, we, b, F, penalty, guard_row, Tm, ones_col):
    acc = jnp.zeros((_SP, F), jnp.float32)
    for h in range(_H):
        fth = ftall[:, h * F:(h + 1) * F]
        fthb = fth.astype(jnp.bfloat16)
        alh = al[h:h + 1, :]
        arh = ar[h:h + 1, :]
        weh = we[h:h + 1, :]
        el = jax.lax.dot_general(fth, alh, (((1,), (1,)), ((), ())),
                                 preferred_element_type=jnp.float32)
        er = jax.lax.dot_general(arh, fth, (((1,), (1,)), ((), ())),
                                 preferred_element_type=jnp.float32)
        eec = jnp.sum(weh * ae[h:h + 1, :])
        lg = _lrelu(el + er + Tm * eec, 0.2) + penalty
        mx = jnp.max(lg, axis=0, keepdims=True)
        ex = jnp.exp(lg - mx)
        den = jnp.sum(ex, axis=0, keepdims=True)
        alpha = ex * (guard_row / den)
        ab = alpha.astype(jnp.bfloat16)
        atb = (alpha * Tm).astype(jnp.bfloat16)
        outh = jax.lax.dot_general(ab, fthb, (((0,), (0,)), ((), ())),
                                   preferred_element_type=jnp.float32)
        eagg = jax.lax.dot_general(atb, ones_col, (((0,), (0,)), ((), ())),
                                   preferred_element_type=jnp.float32)
        outh = outh + eagg * weh + b[h:h + 1, :]
        acc = acc + _lrelu(outh, 0.01)
    return acc * (1.0 / _H)


def _gnn_body(feat_ref, g_ref, t_ref, lng_ref, lnb_ref, w0_ref, al0_ref,
              ar0_ref, ae0_ref, we0_ref, b0_ref, w1_ref, al1_ref, ar1_ref,
              ae1_ref, we1_ref, b1_ref, out_ref):
    lane = jax.lax.broadcasted_iota(jnp.int32, (_SP, 8), 1)
    feat_on = lane < 5
    ones_col = jnp.ones((_SP, 1), jnp.bfloat16)
    for gix in range(_B):
        f = feat_ref[gix]
        fm = jnp.where(feat_on, f, 0.0)
        mu = jnp.sum(fm, axis=1, keepdims=True) * 0.2
        var = jnp.sum(jnp.where(feat_on, (fm - mu) ** 2, 0.0),
                      axis=1, keepdims=True) * 0.2
        nf = (fm - mu) * jax.lax.rsqrt(var + 1e-5) * lng_ref[...] \
            + lnb_ref[...]
        nf = jnp.where(feat_on, nf, 0.0)

        g = g_ref[gix]
        penalty = (g - 1.0) * 1e9
        Tm = t_ref[gix]
        deg = jnp.sum(g, axis=0, keepdims=True)
        guard_row = jnp.where(deg > 0, 1.0, 0.0)

        ft0 = jnp.dot(nf, w0_ref[...], preferred_element_type=jnp.float32)
        h0 = _gat(ft0, al0_ref[...], ar0_ref[...], ae0_ref[...],
                  we0_ref[...], b0_ref[...], _F0, penalty, guard_row,
                  Tm, ones_col)
        ft1 = jnp.dot(h0, w1_ref[...], preferred_element_type=jnp.float32)
        h1 = _gat(ft1, al1_ref[...], ar1_ref[...], ae1_ref[...],
                  we1_ref[...], b1_ref[...], _ED, penalty, guard_row,
                  Tm, ones_col)
        out_ref[gix] = h1


def _final_body(x_ref, w_ref, b_ref, out_ref):
    k = pl.program_id(0)
    acc = jnp.zeros((x_ref.shape[0], _ED), jnp.float32)
    for n in range(8):
        acc = acc + jnp.dot(x_ref[:, n, :], w_ref[n],
                            preferred_element_type=jnp.float32)

    @pl.when(k == 0)
    def _():
        out_ref[...] = acc

    @pl.when(k > 0)
    def _():
        out_ref[...] = out_ref[...] + acc

    @pl.when(k == (_N // 8) - 1)
    def _():
        out_ref[...] = _lrelu(out_ref[...] + b_ref[...], 0.01)


def kernel(Graph, norm_h, norm_L, norm_W, norm_P, norm_N, T, ln_g, ln_b,
           W0, We0, al0, ar0, ae0, b0, W1, We1, al1, ar1, ae1, b1, Wl, bl):
    bs = Graph.shape[0]
    G = Graph.reshape(bs, _J, _N)
    Gp = jnp.pad(G, ((0, 0), (0, _SP - _J), (0, _SP - _N)))
    Tp = jnp.pad(T, ((0, 0), (0, _SP - _J), (0, _SP - _J)))
    other = jnp.concatenate([norm_W, norm_P, norm_N], axis=1)
    jobf = jnp.concatenate(
        [norm_h[:, :, None], norm_L[:, :, None],
         jnp.broadcast_to(other[:, None, :], (bs, _J, 3))], axis=2)
    feats = jnp.pad(jobf, ((0, 0), (0, _SP - _J), (0, 3)))
    lng = jnp.pad(ln_g, (0, 3)).reshape(1, 8)
    lnb = jnp.pad(ln_b, (0, 3)).reshape(1, 8)
    w0 = jnp.pad(W0, ((0, 3), (0, 0)))
    we0 = We0.reshape(_H, _F0)
    b0r = b0.reshape(_H, _F0)
    we1 = We1.reshape(_H, _ED)
    b1r = b1.reshape(_H, _ED)

    h1 = pl.pallas_call(
        _gnn_body,
        grid=(bs // _B,),
        in_specs=[
            pl.BlockSpec((_B, _SP, 8), lambda i: (i, 0, 0)),
            pl.BlockSpec((_B, _SP, _SP), lambda i: (i, 0, 0)),
            pl.BlockSpec((_B, _SP, _SP), lambda i: (i, 0, 0)),
            pl.BlockSpec((1, 8), lambda i: (0, 0)),
            pl.BlockSpec((1, 8), lambda i: (0, 0)),
            pl.BlockSpec((8, _H * _F0), lambda i: (0, 0)),
            pl.BlockSpec((_H, _F0), lambda i: (0, 0)),
            pl.BlockSpec((_H, _F0), lambda i: (0, 0)),
            pl.BlockSpec((_H, _F0), lambda i: (0, 0)),
            pl.BlockSpec((_H, _F0), lambda i: (0, 0)),
            pl.BlockSpec((_H, _F0), lambda i: (0, 0)),
            pl.BlockSpec((_F0, _H * _ED), lambda i: (0, 0)),
            pl.BlockSpec((_H, _ED), lambda i: (0, 0)),
            pl.BlockSpec((_H, _ED), lambda i: (0, 0)),
            pl.BlockSpec((_H, _ED), lambda i: (0, 0)),
            pl.BlockSpec((_H, _ED), lambda i: (0, 0)),
            pl.BlockSpec((_H, _ED), lambda i: (0, 0)),
        ],
        out_specs=pl.BlockSpec((_B, _SP, _SP), lambda i: (i, 0, 0)),
        out_shape=jax.ShapeDtypeStruct((bs, _SP, _SP), jnp.float32),
    )(feats, Gp, Tp, lng, lnb, w0, al0, ar0, ae0, we0, b0r,
      W1, al1, ar1, ae1, we1, b1r)

    Wlr = Wl.reshape(_N, _ED, _ED)
    out = pl.pallas_call(
        _final_body,
        grid=(_N // 8,),
        in_specs=[
            pl.BlockSpec((bs, 8, _ED), lambda k: (0, k, 0)),
            pl.BlockSpec((8, _ED, _ED), lambda k: (k, 0, 0)),
            pl.BlockSpec((1, _ED), lambda k: (0, 0)),
        ],
        out_specs=pl.BlockSpec((bs, _ED), lambda k: (0, 0)),
        out_shape=jax.ShapeDtypeStruct((bs, _ED), jnp.float32),
    )(h1, Wlr, bl.reshape(1, _ED))
    return out


# speedup vs baseline: 1.0634x; 1.0432x over previous
import jax
import jax.numpy as jnp
from jax.experimental import pallas as pl

_J = 100
_M = 20
_N = _J + _M
_H = 3
_F0 = 16
_ED = 128
_SP = 128
_B = 2


def _lrelu(x, s):
    return jnp.maximum(x, s * x)


def _gat(ftall, al, ar, ae, we, b, F, penalty, guard_row, Tm, ones_col):
    acc = jnp.zeros((_SP, F), jnp.float32)
    for h in range(_H):
        fth = ftall[:, h * F:(h + 1) * F]
        fthb = fth.astype(jnp.bfloat16)
        alh = al[h:h + 1, :]
        arh = ar[h:h + 1, :]
        weh = we[h:h + 1, :]
        el = jax.lax.dot_general(fth, alh, (((1,), (1,)), ((), ())),
                                 preferred_element_type=jnp.float32)
        er = jax.lax.dot_general(arh, fth, (((1,), (1,)), ((), ())),
                                 preferred_element_type=jnp.float32)
        eec = jnp.sum(weh * ae[h:h + 1, :])
        lg = _lrelu(el + er + Tm * eec, 0.2) + penalty
        mx = jnp.max(lg, axis=0, keepdims=True)
        ex = jnp.exp(lg - mx)
        den = jnp.sum(ex, axis=0, keepdims=True)
        alpha = ex * (guard_row / den)
        ab = alpha.astype(jnp.bfloat16)
        atb = (alpha * Tm).astype(jnp.bfloat16)
        outh = jax.lax.dot_general(ab, fthb, (((0,), (0,)), ((), ())),
                                   preferred_element_type=jnp.float32)
        eagg = jax.lax.dot_general(atb, ones_col, (((0,), (0,)), ((), ())),
                                   preferred_element_type=jnp.float32)
        outh = outh + eagg * weh + b[h:h + 1, :]
        acc = acc + _lrelu(outh, 0.01)
    return acc * (1.0 / _H)


def _gnn_body(feat_ref, g_ref, t_ref, lng_ref, lnb_ref, w0_ref, al0_ref,
              ar0_ref, ae0_ref, we0_ref, b0_ref, w1_ref, al1_ref, ar1_ref,
              ae1_ref, we1_ref, b1_ref, out_ref):
    lane = jax.lax.broadcasted_iota(jnp.int32, (_SP, 8), 1)
    feat_on = lane < 5
    ones_col = jnp.ones((_SP, 1), jnp.bfloat16)
    for gix in range(_B):
        f = feat_ref[gix]
        fm = jnp.where(feat_on, f, 0.0)
        mu = jnp.sum(fm, axis=1, keepdims=True) * 0.2
        var = jnp.sum(jnp.where(feat_on, (fm - mu) ** 2, 0.0),
                      axis=1, keepdims=True) * 0.2
        nf = (fm - mu) * jax.lax.rsqrt(var + 1e-5) * lng_ref[...] \
            + lnb_ref[...]
        nf = jnp.where(feat_on, nf, 0.0)

        g = jnp.pad(g_ref[gix], ((0, _SP - _J), (0, _SP - _N)))
        penalty = (g - 1.0) * 1e9
        Tm = jnp.pad(t_ref[gix], ((0, _SP - _J), (0, _SP - _J)))
        deg = jnp.sum(g, axis=0, keepdims=True)
        guard_row = jnp.where(deg > 0, 1.0, 0.0)

        ft0 = jnp.dot(nf, w0_ref[...], preferred_element_type=jnp.float32)
        h0 = _gat(ft0, al0_ref[...], ar0_ref[...], ae0_ref[...],
                  we0_ref[...], b0_ref[...], _F0, penalty, guard_row,
                  Tm, ones_col)
        ft1 = jnp.dot(h0, w1_ref[...], preferred_element_type=jnp.float32)
        h1 = _gat(ft1, al1_ref[...], ar1_ref[...], ae1_ref[...],
                  we1_ref[...], b1_ref[...], _ED, penalty, guard_row,
                  Tm, ones_col)
        out_ref[gix] = h1


def _final_body(x_ref, w_ref, b_ref, out_ref):
    k = pl.program_id(0)
    acc = jnp.zeros((x_ref.shape[0], _ED), jnp.float32)
    for n in range(8):
        acc = acc + jnp.dot(x_ref[:, n, :], w_ref[n],
                            preferred_element_type=jnp.float32)

    @pl.when(k == 0)
    def _():
        out_ref[...] = acc

    @pl.when(k > 0)
    def _():
        out_ref[...] = out_ref[...] + acc

    @pl.when(k == (_N // 8) - 1)
    def _():
        out_ref[...] = _lrelu(out_ref[...] + b_ref[...], 0.01)


def kernel(Graph, norm_h, norm_L, norm_W, norm_P, norm_N, T, ln_g, ln_b,
           W0, We0, al0, ar0, ae0, b0, W1, We1, al1, ar1, ae1, b1, Wl, bl):
    bs = Graph.shape[0]
    Gp = Graph.reshape(bs, _J, _N)
    Tp = T
    other = jnp.concatenate([norm_W, norm_P, norm_N], axis=1)
    jobf = jnp.concatenate(
        [norm_h[:, :, None], norm_L[:, :, None],
         jnp.broadcast_to(other[:, None, :], (bs, _J, 3))], axis=2)
    feats = jnp.pad(jobf, ((0, 0), (0, _SP - _J), (0, 3)))
    lng = jnp.pad(ln_g, (0, 3)).reshape(1, 8)
    lnb = jnp.pad(ln_b, (0, 3)).reshape(1, 8)
    w0 = jnp.pad(W0, ((0, 3), (0, 0)))
    we0 = We0.reshape(_H, _F0)
    b0r = b0.reshape(_H, _F0)
    we1 = We1.reshape(_H, _ED)
    b1r = b1.reshape(_H, _ED)

    h1 = pl.pallas_call(
        _gnn_body,
        grid=(bs // _B,),
        in_specs=[
            pl.BlockSpec((_B, _SP, 8), lambda i: (i, 0, 0)),
            pl.BlockSpec((_B, _J, _N), lambda i: (i, 0, 0)),
            pl.BlockSpec((_B, _J, _J), lambda i: (i, 0, 0)),
            pl.BlockSpec((1, 8), lambda i: (0, 0)),
            pl.BlockSpec((1, 8), lambda i: (0, 0)),
            pl.BlockSpec((8, _H * _F0), lambda i: (0, 0)),
            pl.BlockSpec((_H, _F0), lambda i: (0, 0)),
            pl.BlockSpec((_H, _F0), lambda i: (0, 0)),
            pl.BlockSpec((_H, _F0), lambda i: (0, 0)),
            pl.BlockSpec((_H, _F0), lambda i: (0, 0)),
            pl.BlockSpec((_H, _F0), lambda i: (0, 0)),
            pl.BlockSpec((_F0, _H * _ED), lambda i: (0, 0)),
            pl.BlockSpec((_H, _ED), lambda i: (0, 0)),
            pl.BlockSpec((_H, _ED), lambda i: (0, 0)),
            pl.BlockSpec((_H, _ED), lambda i: (0, 0)),
            pl.BlockSpec((_H, _ED), lambda i: (0, 0)),
            pl.BlockSpec((_H, _ED), lambda i: (0, 0)),
        ],
        out_specs=pl.BlockSpec((_B, _SP, _SP), lambda i: (i, 0, 0)),
        out_shape=jax.ShapeDtypeStruct((bs, _SP, _SP), jnp.float32),
    )(feats, Gp, Tp, lng, lnb, w0, al0, ar0, ae0, we0, b0r,
      W1, al1, ar1, ae1, we1, b1r)

    Wlr = Wl.reshape(_N, _ED, _ED)
    out = pl.pallas_call(
        _final_body,
        grid=(_N // 8,),
        in_specs=[
            pl.BlockSpec((bs, 8, _ED), lambda k: (0, k, 0)),
            pl.BlockSpec((8, _ED, _ED), lambda k: (k, 0, 0)),
            pl.BlockSpec((1, _ED), lambda k: (0, 0)),
        ],
        out_specs=pl.BlockSpec((bs, _ED), lambda k: (0, 0)),
        out_shape=jax.ShapeDtypeStruct((bs, _ED), jnp.float32),
    )(h1, Wlr, bl.reshape(1, _ED))
    return out


# speedup vs baseline: 1.1062x; 1.0402x over previous
import jax
import jax.numpy as jnp
from jax.experimental import pallas as pl

_J = 100
_M = 20
_N = _J + _M
_H = 3
_F0 = 16
_ED = 128
_SP = 128
_B = 4


def _lrelu(x, s):
    return jnp.maximum(x, s * x)


def _gat(ftall, al, ar, ae, we, b, F, penalty, guard_row, Tm, ones_col):
    acc = jnp.zeros((_SP, F), jnp.float32)
    for h in range(_H):
        fth = ftall[:, h * F:(h + 1) * F]
        fthb = fth.astype(jnp.bfloat16)
        alh = al[h:h + 1, :]
        arh = ar[h:h + 1, :]
        weh = we[h:h + 1, :]
        el = jax.lax.dot_general(fth, alh, (((1,), (1,)), ((), ())),
                                 preferred_element_type=jnp.float32)
        er = jax.lax.dot_general(arh, fth, (((1,), (1,)), ((), ())),
                                 preferred_element_type=jnp.float32)
        eec = jnp.sum(weh * ae[h:h + 1, :])
        lg = _lrelu(el + er + Tm * eec, 0.2) + penalty
        mx = jnp.max(lg, axis=0, keepdims=True)
        ex = jnp.exp(lg - mx)
        den = jnp.sum(ex, axis=0, keepdims=True)
        alpha = ex * (guard_row / den)
        ab = alpha.astype(jnp.bfloat16)
        atb = (alpha * Tm).astype(jnp.bfloat16)
        outh = jax.lax.dot_general(ab, fthb, (((0,), (0,)), ((), ())),
                                   preferred_element_type=jnp.float32)
        eagg = jax.lax.dot_general(atb, ones_col, (((0,), (0,)), ((), ())),
                                   preferred_element_type=jnp.float32)
        outh = outh + eagg * weh + b[h:h + 1, :]
        acc = acc + _lrelu(outh, 0.01)
    return acc * (1.0 / _H)


def _gnn_body(feat_ref, g_ref, t_ref, lng_ref, lnb_ref, w0_ref, al0_ref,
              ar0_ref, ae0_ref, we0_ref, b0_ref, w1_ref, al1_ref, ar1_ref,
              ae1_ref, we1_ref, b1_ref, out_ref):
    lane = jax.lax.broadcasted_iota(jnp.int32, (_SP, 8), 1)
    feat_on = lane < 5
    ones_col = jnp.ones((_SP, 1), jnp.bfloat16)
    for gix in range(_B):
        f = feat_ref[gix]
        fm = jnp.where(feat_on, f, 0.0)
        mu = jnp.sum(fm, axis=1, keepdims=True) * 0.2
        var = jnp.sum(jnp.where(feat_on, (fm - mu) ** 2, 0.0),
                      axis=1, keepdims=True) * 0.2
        nf = (fm - mu) * jax.lax.rsqrt(var + 1e-5) * lng_ref[...] \
            + lnb_ref[...]
        nf = jnp.where(feat_on, nf, 0.0)

        g = jnp.pad(g_ref[gix], ((0, _SP - _J), (0, _SP - _N)))
        penalty = (g - 1.0) * 1e9
        Tm = jnp.pad(t_ref[gix], ((0, _SP - _J), (0, _SP - _J)))
        deg = jnp.sum(g, axis=0, keepdims=True)
        guard_row = jnp.where(deg > 0, 1.0, 0.0)

        ft0 = jnp.dot(nf, w0_ref[...], preferred_element_type=jnp.float32)
        h0 = _gat(ft0, al0_ref[...], ar0_ref[...], ae0_ref[...],
                  we0_ref[...], b0_ref[...], _F0, penalty, guard_row,
                  Tm, ones_col)
        ft1 = jnp.dot(h0, w1_ref[...], preferred_element_type=jnp.float32)
        h1 = _gat(ft1, al1_ref[...], ar1_ref[...], ae1_ref[...],
                  we1_ref[...], b1_ref[...], _ED, penalty, guard_row,
                  Tm, ones_col)
        out_ref[gix] = h1


def _final_body(x_ref, w_ref, b_ref, out_ref):
    k = pl.program_id(0)
    acc = jnp.zeros((x_ref.shape[0], _ED), jnp.float32)
    for n in range(8):
        acc = acc + jnp.dot(x_ref[:, n, :], w_ref[n],
                            preferred_element_type=jnp.float32)

    @pl.when(k == 0)
    def _():
        out_ref[...] = acc

    @pl.when(k > 0)
    def _():
        out_ref[...] = out_ref[...] + acc

    @pl.when(k == (_N // 8) - 1)
    def _():
        out_ref[...] = _lrelu(out_ref[...] + b_ref[...], 0.01)


def kernel(Graph, norm_h, norm_L, norm_W, norm_P, norm_N, T, ln_g, ln_b,
           W0, We0, al0, ar0, ae0, b0, W1, We1, al1, ar1, ae1, b1, Wl, bl):
    bs = Graph.shape[0]
    Gp = Graph.reshape(bs, _J, _N)
    Tp = T
    other = jnp.concatenate([norm_W, norm_P, norm_N], axis=1)
    jobf = jnp.concatenate(
        [norm_h[:, :, None], norm_L[:, :, None],
         jnp.broadcast_to(other[:, None, :], (bs, _J, 3))], axis=2)
    feats = jnp.pad(jobf, ((0, 0), (0, _SP - _J), (0, 3)))
    lng = jnp.pad(ln_g, (0, 3)).reshape(1, 8)
    lnb = jnp.pad(ln_b, (0, 3)).reshape(1, 8)
    w0 = jnp.pad(W0, ((0, 3), (0, 0)))
    we0 = We0.reshape(_H, _F0)
    b0r = b0.reshape(_H, _F0)
    we1 = We1.reshape(_H, _ED)
    b1r = b1.reshape(_H, _ED)

    h1 = pl.pallas_call(
        _gnn_body,
        grid=(bs // _B,),
        in_specs=[
            pl.BlockSpec((_B, _SP, 8), lambda i: (i, 0, 0)),
            pl.BlockSpec((_B, _J, _N), lambda i: (i, 0, 0)),
            pl.BlockSpec((_B, _J, _J), lambda i: (i, 0, 0)),
            pl.BlockSpec((1, 8), lambda i: (0, 0)),
            pl.BlockSpec((1, 8), lambda i: (0, 0)),
            pl.BlockSpec((8, _H * _F0), lambda i: (0, 0)),
            pl.BlockSpec((_H, _F0), lambda i: (0, 0)),
            pl.BlockSpec((_H, _F0), lambda i: (0, 0)),
            pl.BlockSpec((_H, _F0), lambda i: (0, 0)),
            pl.BlockSpec((_H, _F0), lambda i: (0, 0)),
            pl.BlockSpec((_H, _F0), lambda i: (0, 0)),
            pl.BlockSpec((_F0, _H * _ED), lambda i: (0, 0)),
            pl.BlockSpec((_H, _ED), lambda i: (0, 0)),
            pl.BlockSpec((_H, _ED), lambda i: (0, 0)),
            pl.BlockSpec((_H, _ED), lambda i: (0, 0)),
            pl.BlockSpec((_H, _ED), lambda i: (0, 0)),
            pl.BlockSpec((_H, _ED), lambda i: (0, 0)),
        ],
        out_specs=pl.BlockSpec((_B, _SP, _SP), lambda i: (i, 0, 0)),
        out_shape=jax.ShapeDtypeStruct((bs, _SP, _SP), jnp.float32),
    )(feats, Gp, Tp, lng, lnb, w0, al0, ar0, ae0, we0, b0r,
      W1, al1, ar1, ae1, we1, b1r)

    Wlr = Wl.reshape(_N, _ED, _ED)
    out = pl.pallas_call(
        _final_body,
        grid=(_N // 8,),
        in_specs=[
            pl.BlockSpec((bs, 8, _ED), lambda k: (0, k, 0)),
            pl.BlockSpec((8, _ED, _ED), lambda k: (k, 0, 0)),
            pl.BlockSpec((1, _ED), lambda k: (0, 0)),
        ],
        out_specs=pl.BlockSpec((bs, _ED), lambda k: (0, 0)),
        out_shape=jax.ShapeDtypeStruct((bs, _ED), jnp.float32),
    )(h1, Wlr, bl.reshape(1, _ED))
    return out


# speedup vs baseline: 1.1132x; 1.0063x over previous
import jax
import jax.numpy as jnp
from jax.experimental import pallas as pl

_J = 100
_M = 20
_N = _J + _M
_H = 3
_F0 = 16
_ED = 128
_SP = 128
_B = 8


def _lrelu(x, s):
    return jnp.maximum(x, s * x)


def _gat(ftall, al, ar, ae, we, b, F, penalty, guard_row, Tm, ones_col):
    acc = jnp.zeros((_SP, F), jnp.float32)
    for h in range(_H):
        fth = ftall[:, h * F:(h + 1) * F]
        fthb = fth.astype(jnp.bfloat16)
        alh = al[h:h + 1, :]
        arh = ar[h:h + 1, :]
        weh = we[h:h + 1, :]
        el = jax.lax.dot_general(fth, alh, (((1,), (1,)), ((), ())),
                                 preferred_element_type=jnp.float32)
        er = jax.lax.dot_general(arh, fth, (((1,), (1,)), ((), ())),
                                 preferred_element_type=jnp.float32)
        eec = jnp.sum(weh * ae[h:h + 1, :])
        lg = _lrelu(el + er + Tm * eec, 0.2) + penalty
        mx = jnp.max(lg, axis=0, keepdims=True)
        ex = jnp.exp(lg - mx)
        den = jnp.sum(ex, axis=0, keepdims=True)
        alpha = ex * (guard_row / den)
        ab = alpha.astype(jnp.bfloat16)
        atb = (alpha * Tm).astype(jnp.bfloat16)
        outh = jax.lax.dot_general(ab, fthb, (((0,), (0,)), ((), ())),
                                   preferred_element_type=jnp.float32)
        eagg = jax.lax.dot_general(atb, ones_col, (((0,), (0,)), ((), ())),
                                   preferred_element_type=jnp.float32)
        outh = outh + eagg * weh + b[h:h + 1, :]
        acc = acc + _lrelu(outh, 0.01)
    return acc * (1.0 / _H)


def _gnn_body(feat_ref, g_ref, t_ref, lng_ref, lnb_ref, w0_ref, al0_ref,
              ar0_ref, ae0_ref, we0_ref, b0_ref, w1_ref, al1_ref, ar1_ref,
              ae1_ref, we1_ref, b1_ref, out_ref):
    lane = jax.lax.broadcasted_iota(jnp.int32, (_SP, 8), 1)
    feat_on = lane < 5
    ones_col = jnp.ones((_SP, 1), jnp.bfloat16)
    for gix in range(_B):
        f = feat_ref[gix]
        fm = jnp.where(feat_on, f, 0.0)
        mu = jnp.sum(fm, axis=1, keepdims=True) * 0.2
        var = jnp.sum(jnp.where(feat_on, (fm - mu) ** 2, 0.0),
                      axis=1, keepdims=True) * 0.2
        nf = (fm - mu) * jax.lax.rsqrt(var + 1e-5) * lng_ref[...] \
            + lnb_ref[...]
        nf = jnp.where(feat_on, nf, 0.0)

        g = jnp.pad(g_ref[gix], ((0, _SP - _J), (0, _SP - _N)))
        penalty = (g - 1.0) * 1e9
        Tm = jnp.pad(t_ref[gix], ((0, _SP - _J), (0, _SP - _J)))
        deg = jnp.sum(g, axis=0, keepdims=True)
        guard_row = jnp.where(deg > 0, 1.0, 0.0)

        ft0 = jnp.dot(nf, w0_ref[...], preferred_element_type=jnp.float32)
        h0 = _gat(ft0, al0_ref[...], ar0_ref[...], ae0_ref[...],
                  we0_ref[...], b0_ref[...], _F0, penalty, guard_row,
                  Tm, ones_col)
        ft1 = jnp.dot(h0, w1_ref[...], preferred_element_type=jnp.float32)
        h1 = _gat(ft1, al1_ref[...], ar1_ref[...], ae1_ref[...],
                  we1_ref[...], b1_ref[...], _ED, penalty, guard_row,
                  Tm, ones_col)
        out_ref[gix] = h1


def _final_body(x_ref, w_ref, b_ref, out_ref):
    k = pl.program_id(0)
    acc = jnp.zeros((x_ref.shape[0], _ED), jnp.float32)
    for n in range(8):
        acc = acc + jnp.dot(x_ref[:, n, :], w_ref[n],
                            preferred_element_type=jnp.float32)

    @pl.when(k == 0)
    def _():
        out_ref[...] = acc

    @pl.when(k > 0)
    def _():
        out_ref[...] = out_ref[...] + acc

    @pl.when(k == (_N // 8) - 1)
    def _():
        out_ref[...] = _lrelu(out_ref[...] + b_ref[...], 0.01)


def kernel(Graph, norm_h, norm_L, norm_W, norm_P, norm_N, T, ln_g, ln_b,
           W0, We0, al0, ar0, ae0, b0, W1, We1, al1, ar1, ae1, b1, Wl, bl):
    bs = Graph.shape[0]
    Gp = Graph.reshape(bs, _J, _N)
    Tp = T
    other = jnp.concatenate([norm_W, norm_P, norm_N], axis=1)
    jobf = jnp.concatenate(
        [norm_h[:, :, None], norm_L[:, :, None],
         jnp.broadcast_to(other[:, None, :], (bs, _J, 3))], axis=2)
    feats = jnp.pad(jobf, ((0, 0), (0, _SP - _J), (0, 3)))
    lng = jnp.pad(ln_g, (0, 3)).reshape(1, 8)
    lnb = jnp.pad(ln_b, (0, 3)).reshape(1, 8)
    w0 = jnp.pad(W0, ((0, 3), (0, 0)))
    we0 = We0.reshape(_H, _F0)
    b0r = b0.reshape(_H, _F0)
    we1 = We1.reshape(_H, _ED)
    b1r = b1.reshape(_H, _ED)

    h1 = pl.pallas_call(
        _gnn_body,
        grid=(bs // _B,),
        in_specs=[
            pl.BlockSpec((_B, _SP, 8), lambda i: (i, 0, 0)),
            pl.BlockSpec((_B, _J, _N), lambda i: (i, 0, 0)),
            pl.BlockSpec((_B, _J, _J), lambda i: (i, 0, 0)),
            pl.BlockSpec((1, 8), lambda i: (0, 0)),
            pl.BlockSpec((1, 8), lambda i: (0, 0)),
            pl.BlockSpec((8, _H * _F0), lambda i: (0, 0)),
            pl.BlockSpec((_H, _F0), lambda i: (0, 0)),
            pl.BlockSpec((_H, _F0), lambda i: (0, 0)),
            pl.BlockSpec((_H, _F0), lambda i: (0, 0)),
            pl.BlockSpec((_H, _F0), lambda i: (0, 0)),
            pl.BlockSpec((_H, _F0), lambda i: (0, 0)),
            pl.BlockSpec((_F0, _H * _ED), lambda i: (0, 0)),
            pl.BlockSpec((_H, _ED), lambda i: (0, 0)),
            pl.BlockSpec((_H, _ED), lambda i: (0, 0)),
            pl.BlockSpec((_H, _ED), lambda i: (0, 0)),
            pl.BlockSpec((_H, _ED), lambda i: (0, 0)),
            pl.BlockSpec((_H, _ED), lambda i: (0, 0)),
        ],
        out_specs=pl.BlockSpec((_B, _SP, _SP), lambda i: (i, 0, 0)),
        out_shape=jax.ShapeDtypeStruct((bs, _SP, _SP), jnp.float32),
    )(feats, Gp, Tp, lng, lnb, w0, al0, ar0, ae0, we0, b0r,
      W1, al1, ar1, ae1, we1, b1r)

    Wlr = Wl.reshape(_N, _ED, _ED)
    out = pl.pallas_call(
        _final_body,
        grid=(_N // 8,),
        in_specs=[
            pl.BlockSpec((bs, 8, _ED), lambda k: (0, k, 0)),
            pl.BlockSpec((8, _ED, _ED), lambda k: (k, 0, 0)),
            pl.BlockSpec((1, _ED), lambda k: (0, 0)),
        ],
        out_specs=pl.BlockSpec((bs, _ED), lambda k: (0, 0)),
        out_shape=jax.ShapeDtypeStruct((bs, _ED), jnp.float32),
    )(h1, Wlr, bl.reshape(1, _ED))
    return out
